# baseline (speedup 1.0000x reference)
import functools

import jax
import jax.numpy as jnp
from jax import lax
from jax.experimental import pallas as pl
from jax.experimental.pallas import tpu as pltpu
from jax.experimental.pallas import tpu_sc as plsc

N_STIMULI = 30
N_DIM = 10
BATCH = 16384

NPAD = 32
_NC, _NS, _L = 2, 16, 16
_NW = _NC * _NS
_BPW = BATCH // _NW
_EPS = NPAD * NPAD // _NS

_EMB = (N_STIMULI + 1) * N_DIM


def _rsqrt_newton(x):
    bits = plsc.bitcast(x, jnp.int32)
    y = plsc.bitcast(0x5F3759DF - (bits >> 1), jnp.float32)
    for _ in range(3):
        y = y * (1.5 - 0.5 * x * y * y)
    return y


@functools.lru_cache(maxsize=1)
def _make_sc_kernel():
    mesh = plsc.VectorSubcoreMesh(core_axis_name="c", subcore_axis_name="s")

    @functools.partial(
        pl.kernel,
        mesh=mesh,
        out_type=jax.ShapeDtypeStruct((BATCH,), jnp.float32),
        scratch_types=[
            pltpu.VMEM((2 * _BPW,), jnp.int32),
            pltpu.VMEM((_EMB,), jnp.float32),
            pltpu.VMEM((_L,), jnp.float32),
            pltpu.VMEM((_EPS,), jnp.float32),
            pltpu.VMEM((NPAD * NPAD,), jnp.float32),
            pltpu.VMEM((_BPW,), jnp.float32),
            pltpu.VMEM_SHARED((NPAD * NPAD,), jnp.float32),
            pltpu.SemaphoreType.DMA,
            pltpu.SemaphoreType.DMA,
            pltpu.SemaphoreType.DMA,
        ],
        compiler_params=pltpu.CompilerParams(needs_layout_passes=False),
    )
    def _sc_kernel(idx_hbm, emb_hbm, pw_hbm, out_hbm,
                   idx_v, emb_v, pw_v, slice_v, tab_v, out_v, shared,
                   sem0, sem1, sem2):
        cid = lax.axis_index("c")
        sid = lax.axis_index("s")
        wid = sid * _NC + cid
        base = wid * _BPW

        idx_cp = pltpu.async_copy(idx_hbm.at[pl.ds(2 * base, 2 * _BPW)],
                                  idx_v, sem0)
        emb_cp = pltpu.async_copy(emb_hbm, emb_v, sem1)
        pw_cp = pltpu.async_copy(pw_hbm, pw_v, sem2)

        lane = lax.iota(jnp.int32, _L)

        def _splat(vec, pos):
            return jnp.sum(jnp.where(lane == pos, vec, 0.0))

        pw_cp.wait()
        emb_cp.wait()
        pv = pw_v[pl.ds(0, _L)]
        wks = [_splat(pv, k) for k in range(N_DIM)]
        lower = _splat(pv, N_DIM)
        upper = _splat(pv, N_DIM + 1)
        midpt = _splat(pv, N_DIM + 2)
        rate = _splat(pv, N_DIM + 3)
        for v in range(_EPS // _L):
            d_off = sid * 2 + (v >> 1)
            i_raw = lane + (v & 1) * _L
            j_raw = (i_raw + d_off) & (NPAD - 1)
            iv = jnp.minimum(i_raw, N_STIMULI)
            jv = jnp.minimum(j_raw, N_STIMULI)
            d2 = jnp.zeros((_L,), jnp.float32)
            for k in range(N_DIM):
                a = plsc.load_gather(emb_v, [iv * N_DIM + k])
                b = plsc.load_gather(emb_v, [jv * N_DIM + k])
                wk = wks[k]
                diff = a - b
                d2 = d2 + wk * diff * diff
            d = d2 * _rsqrt_newton(d2)
            s = jnp.exp(-3.0 * d)
            slice_v[pl.ds(v * _L, _L)] = (
                lower + (upper - lower) / (1.0 + jnp.exp(-rate * (s - midpt))))
        pltpu.sync_copy(slice_v, shared.at[pl.ds(sid * _EPS, _EPS)])
        plsc.subcore_barrier()
        pltpu.sync_copy(shared, tab_v)

        idx_cp.wait()
        for m in range(_BPW // _L):
            off = (m // 8) * 256 + (m % 8) * _L
            iv = idx_v[pl.ds(off, _L)]
            jv = idx_v[pl.ds(off + 128, _L)]
            tpos = ((jv - iv) & (NPAD - 1)) * NPAD + iv
            out_v[pl.ds(m * _L, _L)] = plsc.load_gather(tab_v, [tpos])
        pltpu.sync_copy(out_v, out_hbm.at[pl.ds(base, _BPW)])

    return _sc_kernel


def kernel(rate2_stimulus_set, embedding, w, lower, upper, midpoint, rate):
    pw = jnp.concatenate([
        w.astype(jnp.float32),
        jnp.stack([lower, upper, midpoint, rate]).astype(jnp.float32),
        jnp.zeros((_L - N_DIM - 4,), jnp.float32),
    ])
    idx_flat = rate2_stimulus_set.reshape(128, 128, 2).transpose(0, 2, 1).reshape(-1)
    y = _make_sc_kernel()(idx_flat, embedding.reshape(-1), pw)
    return y.reshape(BATCH, 1)


# speedup vs baseline: 1.0242x; 1.0242x over previous
import functools

import jax
import jax.numpy as jnp
from jax import lax
from jax.experimental import pallas as pl
from jax.experimental.pallas import tpu as pltpu
from jax.experimental.pallas import tpu_sc as plsc

N_STIMULI = 30
N_DIM = 10
BATCH = 16384

NPAD = 32
_NC, _NS, _L = 2, 16, 16
_NW = _NC * _NS
_BPW = BATCH // _NW
_EPS = NPAD * NPAD // _NS

_EMB = (N_STIMULI + 1) * N_DIM
_W0 = _EMB
_S0 = _EMB + N_DIM
_PACK = 384
_SM0 = 304
_SMW = 32


def _rsqrt_newton(x):
    bits = plsc.bitcast(x, jnp.int32)
    y = plsc.bitcast(0x5F3759DF - (bits >> 1), jnp.float32)
    for _ in range(3):
        y = y * (1.5 - 0.5 * x * y * y)
    return y


@functools.lru_cache(maxsize=1)
def _make_sc_kernel():
    mesh = plsc.VectorSubcoreMesh(core_axis_name="c", subcore_axis_name="s")

    @functools.partial(
        pl.kernel,
        mesh=mesh,
        out_type=jax.ShapeDtypeStruct((BATCH,), jnp.float32),
        scratch_types=[
            pltpu.VMEM((2 * _BPW,), jnp.int32),
            pltpu.VMEM((_PACK,), jnp.float32),
            pltpu.VMEM((_EPS,), jnp.float32),
            pltpu.VMEM((NPAD * NPAD,), jnp.float32),
            pltpu.VMEM((_BPW,), jnp.float32),
            pltpu.VMEM_SHARED((NPAD * NPAD,), jnp.float32),
            pltpu.SemaphoreType.DMA,
            pltpu.SemaphoreType.DMA,
        ],
        compiler_params=pltpu.CompilerParams(needs_layout_passes=False),
    )
    def _sc_kernel(idx_hbm, pack_hbm, out_hbm,
                   idx_v, data_v, slice_v, tab_v, out_v, shared,
                   sem0, sem1):
        cid = lax.axis_index("c")
        sid = lax.axis_index("s")
        wid = sid * _NC + cid
        base = wid * _BPW

        idx_cp = pltpu.async_copy(idx_hbm.at[pl.ds(2 * base, 2 * _BPW)],
                                  idx_v, sem0)
        pltpu.sync_copy(pack_hbm, data_v)

        lane = lax.iota(jnp.int32, _L)

        def _splat(vec, pos):
            return jnp.sum(jnp.where(lane == pos, vec, 0.0))

        pv1 = data_v[pl.ds(_SM0, _L)]
        pv2 = data_v[pl.ds(_SM0 + _L, _L)]
        wks = [_splat(pv1, _W0 - _SM0 + k) for k in range(N_DIM)]
        lower = _splat(pv2, 0)
        upper = _splat(pv2, 1)
        midpt = _splat(pv2, 2)
        rate = _splat(pv2, 3)
        for v in range(_EPS // _L):
            d_off = sid * 2 + (v >> 1)
            i_raw = lane + (v & 1) * _L
            j_raw = (i_raw + d_off) & (NPAD - 1)
            iv = jnp.minimum(i_raw, N_STIMULI)
            jv = jnp.minimum(j_raw, N_STIMULI)
            d2 = jnp.zeros((_L,), jnp.float32)
            for k in range(N_DIM):
                a = plsc.load_gather(data_v, [iv * N_DIM + k])
                b = plsc.load_gather(data_v, [jv * N_DIM + k])
                wk = wks[k]
                diff = a - b
                d2 = d2 + wk * diff * diff
            d = d2 * _rsqrt_newton(d2)
            s = jnp.exp(-3.0 * d)
            slice_v[pl.ds(v * _L, _L)] = (
                lower + (upper - lower) / (1.0 + jnp.exp(-rate * (s - midpt))))
        pltpu.sync_copy(slice_v, shared.at[pl.ds(sid * _EPS, _EPS)])
        plsc.subcore_barrier()
        pltpu.sync_copy(shared, tab_v)

        idx_cp.wait()
        out_cps = []
        for c in range(4):
            def _lookup(mm, _, c=c):
                off = c * 256 + mm * _L
                iv = idx_v[pl.ds(off, _L)]
                jv = idx_v[pl.ds(off + 128, _L)]
                tpos = ((jv - iv) & (NPAD - 1)) * NPAD + iv
                out_v[pl.ds(c * 128 + mm * _L, _L)] = plsc.load_gather(
                    tab_v, [tpos])
                return 0

            lax.fori_loop(0, 8, _lookup, 0)
            out_cps.append(pltpu.async_copy(
                out_v.at[pl.ds(c * 128, 128)],
                out_hbm.at[pl.ds(base + c * 128, 128)], sem1))
        for cp in out_cps:
            cp.wait()

    return _sc_kernel


def kernel(rate2_stimulus_set, embedding, w, lower, upper, midpoint, rate):
    pack = jnp.concatenate([
        embedding.reshape(-1),
        w.astype(jnp.float32),
        jnp.stack([lower, upper, midpoint, rate]).astype(jnp.float32),
        jnp.zeros((_PACK - _S0 - 4,), jnp.float32),
    ])
    idx_flat = rate2_stimulus_set.reshape(128, 128, 2).transpose(0, 2, 1).reshape(-1)
    y = _make_sc_kernel()(idx_flat, pack)
    return y.reshape(BATCH, 1)
